# baseline (speedup 1.0000x reference)
import functools

import jax
import jax.numpy as jnp
from jax import lax
from jax.experimental import pallas as pl
from jax.experimental.pallas import tpu as pltpu
from jax.experimental.pallas import tpu_sc as plsc

B = 4096
CON = 13
CAT = 26
V = 100000
D = 32
ROWS = 1 + CON + CAT

_INFO = plsc.get_sparse_core_info()
NC, NS, L = _INFO.num_cores, _INFO.num_subcores, _INFO.num_lanes
NW = NC * NS
BPW = B // NW
NG = BPW // L


def _make_chunk(f_lo, nf, with_dense):
    drows = (1 + CON) * D if with_dense else 0
    orows = drows + nf * D

    def body(tab_hbm, xcat_hbm, xcon_hbm, wcon_hbm, bcon_hbm, cls_hbm,
             out_hbm,
             idx_v, gbuf_v, dbuf_v, xc_v, w_v, bb_v, cls_v, gsem):
        wid = lax.axis_index("s") * NC + lax.axis_index("c")
        b0 = pl.multiple_of(wid * BPW, BPW)

        def field_body(f, carry):
            pltpu.sync_copy(xcat_hbm.at[f_lo + f, pl.ds(b0, BPW)], idx_v)
            gathers = [
                pltpu.async_copy(tab_hbm.at[f * D + d].at[idx_v],
                                 gbuf_v.at[d], gsem)
                for d in range(D)
            ]
            for g in gathers:
                g.wait()
            r0 = pl.multiple_of(drows + f * D, D)
            pltpu.sync_copy(gbuf_v, out_hbm.at[pl.ds(r0, D), pl.ds(b0, BPW)])
            return carry

        lax.fori_loop(0, nf, field_body, 0)

        if with_dense:
            for j in range(CON):
                pltpu.sync_copy(xcon_hbm.at[j, pl.ds(b0, BPW)], xc_v.at[j])
            pltpu.sync_copy(wcon_hbm, w_v)
            pltpu.sync_copy(bcon_hbm, bb_v)
            pltpu.sync_copy(cls_hbm, cls_v)

            for d in range(D):
                sp = cls_v[pl.ds(d * L, L)]
                for g in range(NG):
                    dbuf_v[d, pl.ds(g * L, L)] = sp
            pltpu.sync_copy(dbuf_v, out_hbm.at[pl.ds(0, D), pl.ds(b0, BPW)])

            def con_body(j, carry):
                xg = [xc_v[j, pl.ds(g * L, L)] for g in range(NG)]
                jo = pl.multiple_of(j * D * L, D * L)
                for d in range(D):
                    w_s = w_v[pl.ds(jo + d * L, L)]
                    b_s = bb_v[pl.ds(jo + d * L, L)]
                    for g in range(NG):
                        dbuf_v[d, pl.ds(g * L, L)] = jnp.maximum(
                            xg[g] * w_s + b_s, 0.0)
                r0 = pl.multiple_of((1 + j) * D, D)
                pltpu.sync_copy(
                    dbuf_v, out_hbm.at[pl.ds(r0, D), pl.ds(b0, BPW)])
                return carry

            lax.fori_loop(0, CON, con_body, 0)

    return functools.partial(
        pl.kernel,
        out_type=jax.ShapeDtypeStruct((orows, B), jnp.float32),
        mesh=plsc.VectorSubcoreMesh(core_axis_name="c", subcore_axis_name="s"),
        compiler_params=pltpu.CompilerParams(use_tc_tiling_on_sc=False),
        scratch_types=[
            pltpu.VMEM((BPW,), jnp.int32),
            pltpu.VMEM((D, BPW), jnp.float32),
            pltpu.VMEM((D, BPW), jnp.float32),
            pltpu.VMEM((CON, BPW), jnp.float32),
            pltpu.VMEM((CON * D * L,), jnp.float32),
            pltpu.VMEM((CON * D * L,), jnp.float32),
            pltpu.VMEM((D * L,), jnp.float32),
            pltpu.SemaphoreType.DMA,
        ],
    )(body)


_SPLITS = [(0, 7, True), (7, 7, False), (14, 6, False), (20, 6, False)]
_CHUNKS = [_make_chunk(*s) for s in _SPLITS]


def kernel(x_con, x_cat, cls, W_con, b_con, tables):
    tabT = jnp.transpose(tables, (0, 2, 1)).reshape(CAT * D, V)
    xcatT = x_cat.T
    xconT = x_con.T
    wsp = jnp.broadcast_to(W_con[:, :, None], (CON, D, L)).reshape(-1)
    bsp = jnp.broadcast_to(b_con[:, :, None], (CON, D, L)).reshape(-1)
    csp = jnp.broadcast_to(cls[0, :, None], (D, L)).reshape(-1)
    parts = [
        fn(tabT[f_lo * D:(f_lo + nf) * D], xcatT, xconT, wsp, bsp, csp)
        for fn, (f_lo, nf, _) in zip(_CHUNKS, _SPLITS)
    ]
    out = jnp.concatenate(parts, axis=0)
    return out.reshape(ROWS, D, B).transpose(2, 0, 1)


# speedup vs baseline: 1.2230x; 1.2230x over previous
import functools

import jax
import jax.numpy as jnp
from jax import lax
from jax.experimental import pallas as pl
from jax.experimental.pallas import tpu as pltpu
from jax.experimental.pallas import tpu_sc as plsc

B = 4096
CON = 13
CAT = 26
V = 100000
D = 32
ROWS = 1 + CON + CAT

_INFO = plsc.get_sparse_core_info()
NC, NS, L = _INFO.num_cores, _INFO.num_subcores, _INFO.num_lanes
NW = NC * NS
BPW = B // NW
NG = BPW // L


def _splat(vec, lane):
    return jnp.zeros((L,), vec.dtype) + vec[lane]


def _sc_kernel(tab_hbm, xcat_hbm, xcon_hbm, wcon_hbm, bcon_hbm, cls_hbm,
               out_hbm,
               idx_v, gbuf_v, dbuf_v, xc_v, w_v, bb_v, cls_v, gsem):
    wid = lax.axis_index("s") * NC + lax.axis_index("c")
    b0 = pl.multiple_of(wid * BPW, BPW)

    def fire_body(f, carry):
        pltpu.sync_copy(xcat_hbm.at[f, pl.ds(b0, BPW)], idx_v.at[f])
        for d in range(D):
            pltpu.async_copy(tab_hbm.at[f * D + d].at[idx_v.at[f]],
                             gbuf_v.at[f * D + d], gsem)
        return carry

    lax.fori_loop(0, CAT, fire_body, 0)

    def drain_body(f, carry):
        pltpu.make_async_copy(tab_hbm.at[pl.ds(0, D), pl.ds(0, BPW)],
                              gbuf_v.at[pl.ds(0, D)], gsem).wait()
        return carry

    lax.fori_loop(0, CAT, drain_body, 0)

    def write_body(f, carry):
        fo = pl.multiple_of(f * D, D)
        r0 = pl.multiple_of((1 + CON + f) * D, D)
        pltpu.sync_copy(gbuf_v.at[pl.ds(fo, D)],
                        out_hbm.at[pl.ds(r0, D), pl.ds(b0, BPW)])
        return carry

    lax.fori_loop(0, CAT, write_body, 0)

    for j in range(CON):
        pltpu.sync_copy(xcon_hbm.at[j, pl.ds(b0, BPW)], xc_v.at[j])
    pltpu.sync_copy(cls_hbm, cls_v)

    for d in range(D):
        sp = cls_v[pl.ds(d * L, L)]
        for g in range(NG):
            dbuf_v[d, pl.ds(g * L, L)] = sp
    pltpu.sync_copy(dbuf_v, out_hbm.at[pl.ds(0, D), pl.ds(b0, BPW)])

    def con_body(j, carry):
        xg = [xc_v[j, pl.ds(g * L, L)] for g in range(NG)]
        jo = pl.multiple_of(j * D * L, D * L)
        pltpu.sync_copy(wcon_hbm.at[pl.ds(jo, D * L)], w_v)
        pltpu.sync_copy(bcon_hbm.at[pl.ds(jo, D * L)], bb_v)
        for d in range(D):
            w_s = w_v[pl.ds(d * L, L)]
            b_s = bb_v[pl.ds(d * L, L)]
            for g in range(NG):
                dbuf_v[d, pl.ds(g * L, L)] = jnp.maximum(
                    xg[g] * w_s + b_s, 0.0)
        r0 = pl.multiple_of((1 + j) * D, D)
        pltpu.sync_copy(dbuf_v, out_hbm.at[pl.ds(r0, D), pl.ds(b0, BPW)])
        return carry

    lax.fori_loop(0, CON, con_body, 0)


@functools.partial(
    pl.kernel,
    out_type=jax.ShapeDtypeStruct((ROWS * D, B), jnp.float32),
    mesh=plsc.VectorSubcoreMesh(core_axis_name="c", subcore_axis_name="s"),
    compiler_params=pltpu.CompilerParams(use_tc_tiling_on_sc=False),
    scratch_types=[
        pltpu.VMEM((CAT, BPW), jnp.int32),
        pltpu.VMEM((CAT * D, BPW), jnp.float32),
        pltpu.VMEM((D, BPW), jnp.float32),
        pltpu.VMEM((CON, BPW), jnp.float32),
        pltpu.VMEM((D * L,), jnp.float32),
        pltpu.VMEM((D * L,), jnp.float32),
        pltpu.VMEM((D * L,), jnp.float32),
        pltpu.SemaphoreType.DMA,
    ],
)
def _graph_embed(*refs):
    _sc_kernel(*refs)


def kernel(x_con, x_cat, cls, W_con, b_con, tables):
    tabT = jnp.transpose(tables, (0, 2, 1)).reshape(CAT * D, V)
    wsp = jnp.broadcast_to(W_con[:, :, None], (CON, D, L)).reshape(-1)
    bsp = jnp.broadcast_to(b_con[:, :, None], (CON, D, L)).reshape(-1)
    csp = jnp.broadcast_to(cls[0, :, None], (D, L)).reshape(-1)
    out = _graph_embed(tabT, x_cat.T, x_con.T, wsp, bsp, csp)
    return out.reshape(ROWS, D, B).transpose(2, 0, 1)


# speedup vs baseline: 1.2230x; 1.0000x over previous
import functools

import jax
import jax.numpy as jnp
from jax import lax
from jax.experimental import pallas as pl
from jax.experimental.pallas import tpu as pltpu
from jax.experimental.pallas import tpu_sc as plsc

B = 4096
CON = 13
CAT = 26
V = 100000
D = 32
ROWS = 1 + CON + CAT

_INFO = plsc.get_sparse_core_info()
NC, NS, L = _INFO.num_cores, _INFO.num_subcores, _INFO.num_lanes
NW = NC * NS
BPW = B // NW
NG = BPW // L


def _splat(vec, lane):
    return jnp.zeros((L,), vec.dtype) + vec[lane]


def _sc_kernel(tab_hbm, xcat_hbm, xcon_hbm, wcon_hbm, bcon_hbm, cls_hbm,
               out_hbm,
               idx_v, gbuf_v, dbuf_v, xc_v, w_v, bb_v, cls_v, gsem):
    wid = lax.axis_index("s") * NC + lax.axis_index("c")
    b0 = pl.multiple_of(wid * BPW, BPW)

    def fire_body(f, carry):
        pltpu.sync_copy(xcat_hbm.at[f, pl.ds(b0, BPW)], idx_v.at[f])
        for d in range(D):
            pltpu.async_copy(tab_hbm.at[f, d].at[idx_v.at[f]],
                             gbuf_v.at[f * D + d], gsem)
        return carry

    lax.fori_loop(0, CAT, fire_body, 0)

    def drain_body(f, carry):
        pltpu.make_async_copy(tab_hbm.at[0, pl.ds(0, D), pl.ds(0, BPW)],
                              gbuf_v.at[pl.ds(0, D)], gsem).wait()
        return carry

    lax.fori_loop(0, CAT, drain_body, 0)

    def write_body(f, carry):
        fo = pl.multiple_of(f * D, D)
        r0 = pl.multiple_of((1 + CON + f) * D, D)
        pltpu.sync_copy(gbuf_v.at[pl.ds(fo, D)],
                        out_hbm.at[pl.ds(r0, D), pl.ds(b0, BPW)])
        return carry

    lax.fori_loop(0, CAT, write_body, 0)

    for j in range(CON):
        pltpu.sync_copy(xcon_hbm.at[j, pl.ds(b0, BPW)], xc_v.at[j])
    pltpu.sync_copy(cls_hbm, cls_v)

    for d in range(D):
        sp = cls_v[pl.ds(d * L, L)]
        for g in range(NG):
            dbuf_v[d, pl.ds(g * L, L)] = sp
    pltpu.sync_copy(dbuf_v, out_hbm.at[pl.ds(0, D), pl.ds(b0, BPW)])

    def con_body(j, carry):
        xg = [xc_v[j, pl.ds(g * L, L)] for g in range(NG)]
        jo = pl.multiple_of(j * D * L, D * L)
        pltpu.sync_copy(wcon_hbm.at[pl.ds(jo, D * L)], w_v)
        pltpu.sync_copy(bcon_hbm.at[pl.ds(jo, D * L)], bb_v)
        for d in range(D):
            w_s = w_v[pl.ds(d * L, L)]
            b_s = bb_v[pl.ds(d * L, L)]
            for g in range(NG):
                dbuf_v[d, pl.ds(g * L, L)] = jnp.maximum(
                    xg[g] * w_s + b_s, 0.0)
        r0 = pl.multiple_of((1 + j) * D, D)
        pltpu.sync_copy(dbuf_v, out_hbm.at[pl.ds(r0, D), pl.ds(b0, BPW)])
        return carry

    lax.fori_loop(0, CON, con_body, 0)


@functools.partial(
    pl.kernel,
    out_type=jax.ShapeDtypeStruct((ROWS * D, B), jnp.float32),
    mesh=plsc.VectorSubcoreMesh(core_axis_name="c", subcore_axis_name="s"),
    compiler_params=pltpu.CompilerParams(use_tc_tiling_on_sc=False),
    scratch_types=[
        pltpu.VMEM((CAT, BPW), jnp.int32),
        pltpu.VMEM((CAT * D, BPW), jnp.float32),
        pltpu.VMEM((D, BPW), jnp.float32),
        pltpu.VMEM((CON, BPW), jnp.float32),
        pltpu.VMEM((D * L,), jnp.float32),
        pltpu.VMEM((D * L,), jnp.float32),
        pltpu.VMEM((D * L,), jnp.float32),
        pltpu.SemaphoreType.DMA,
    ],
)
def _graph_embed(*refs):
    _sc_kernel(*refs)


def kernel(x_con, x_cat, cls, W_con, b_con, tables):
    tabT = jnp.transpose(tables, (0, 2, 1))
    wsp = jnp.broadcast_to(W_con[:, :, None], (CON, D, L)).reshape(-1)
    bsp = jnp.broadcast_to(b_con[:, :, None], (CON, D, L)).reshape(-1)
    csp = jnp.broadcast_to(cls[0, :, None], (D, L)).reshape(-1)
    out = _graph_embed(tabT, x_cat.T, x_con.T, wsp, bsp, csp)
    return out.reshape(ROWS, D, B).transpose(2, 0, 1)


# speedup vs baseline: 1.2234x; 1.0003x over previous
import functools

import jax
import jax.numpy as jnp
from jax import lax
from jax.experimental import pallas as pl
from jax.experimental.pallas import tpu as pltpu
from jax.experimental.pallas import tpu_sc as plsc

B = 4096
CON = 13
CAT = 26
V = 100000
D = 32
ROWS = 1 + CON + CAT

_INFO = plsc.get_sparse_core_info()
NC, NS, L = _INFO.num_cores, _INFO.num_subcores, _INFO.num_lanes
NW = NC * NS
BPW = B // NW
NG = BPW // L


def _splat(vec, lane):
    return jnp.zeros((L,), vec.dtype) + vec[lane]


def _sc_kernel(tab_hbm, xcat_hbm, xcon_hbm, wcon_hbm, bcon_hbm, cls_hbm,
               out_hbm,
               idx_v, gbuf_v, dbuf_v, xc_v, w_v, bb_v, cls_v, gsem):
    wid = lax.axis_index("s") * NC + lax.axis_index("c")
    b0 = pl.multiple_of(wid * BPW, BPW)

    def fire_body(f, carry):
        pltpu.sync_copy(xcat_hbm.at[f, pl.ds(b0, BPW)], idx_v.at[f])
        for d in range(D):
            pltpu.async_copy(tab_hbm.at[f, d].at[idx_v.at[f]],
                             gbuf_v.at[f * D + d], gsem)
        return carry

    lax.fori_loop(0, CAT, fire_body, 0)

    for j in range(CON):
        pltpu.sync_copy(xcon_hbm.at[j, pl.ds(b0, BPW)], xc_v.at[j])
    pltpu.sync_copy(cls_hbm, cls_v)

    for d in range(D):
        sp = cls_v[pl.ds(d * L, L)]
        for g in range(NG):
            dbuf_v[d, pl.ds(g * L, L)] = sp
    pltpu.sync_copy(dbuf_v, out_hbm.at[pl.ds(0, D), pl.ds(b0, BPW)])

    def con_body(j, carry):
        xg = [xc_v[j, pl.ds(g * L, L)] for g in range(NG)]
        jo = pl.multiple_of(j * D * L, D * L)
        pltpu.sync_copy(wcon_hbm.at[pl.ds(jo, D * L)], w_v)
        pltpu.sync_copy(bcon_hbm.at[pl.ds(jo, D * L)], bb_v)
        for d in range(D):
            w_s = w_v[pl.ds(d * L, L)]
            b_s = bb_v[pl.ds(d * L, L)]
            for g in range(NG):
                dbuf_v[d, pl.ds(g * L, L)] = jnp.maximum(
                    xg[g] * w_s + b_s, 0.0)
        r0 = pl.multiple_of((1 + j) * D, D)
        pltpu.sync_copy(dbuf_v, out_hbm.at[pl.ds(r0, D), pl.ds(b0, BPW)])
        return carry

    lax.fori_loop(0, CON, con_body, 0)

    def drain_body(f, carry):
        pltpu.make_async_copy(tab_hbm.at[0, pl.ds(0, D), pl.ds(0, BPW)],
                              gbuf_v.at[pl.ds(0, D)], gsem).wait()
        return carry

    lax.fori_loop(0, CAT, drain_body, 0)

    def write_body(f, carry):
        fo = pl.multiple_of(f * D, D)
        r0 = pl.multiple_of((1 + CON + f) * D, D)
        pltpu.sync_copy(gbuf_v.at[pl.ds(fo, D)],
                        out_hbm.at[pl.ds(r0, D), pl.ds(b0, BPW)])
        return carry

    lax.fori_loop(0, CAT, write_body, 0)


@functools.partial(
    pl.kernel,
    out_type=jax.ShapeDtypeStruct((ROWS * D, B), jnp.float32),
    mesh=plsc.VectorSubcoreMesh(core_axis_name="c", subcore_axis_name="s"),
    compiler_params=pltpu.CompilerParams(use_tc_tiling_on_sc=False),
    scratch_types=[
        pltpu.VMEM((CAT, BPW), jnp.int32),
        pltpu.VMEM((CAT * D, BPW), jnp.float32),
        pltpu.VMEM((D, BPW), jnp.float32),
        pltpu.VMEM((CON, BPW), jnp.float32),
        pltpu.VMEM((D * L,), jnp.float32),
        pltpu.VMEM((D * L,), jnp.float32),
        pltpu.VMEM((D * L,), jnp.float32),
        pltpu.SemaphoreType.DMA,
    ],
)
def _graph_embed(*refs):
    _sc_kernel(*refs)


def kernel(x_con, x_cat, cls, W_con, b_con, tables):
    tabT = jnp.transpose(tables, (0, 2, 1))
    wsp = jnp.broadcast_to(W_con[:, :, None], (CON, D, L)).reshape(-1)
    bsp = jnp.broadcast_to(b_con[:, :, None], (CON, D, L)).reshape(-1)
    csp = jnp.broadcast_to(cls[0, :, None], (D, L)).reshape(-1)
    out = _graph_embed(tabT, x_cat.T, x_con.T, wsp, bsp, csp)
    return out.reshape(ROWS, D, B).transpose(2, 0, 1)


# speedup vs baseline: 1.2243x; 1.0007x over previous
import functools

import jax
import jax.numpy as jnp
from jax import lax
from jax.experimental import pallas as pl
from jax.experimental.pallas import tpu as pltpu
from jax.experimental.pallas import tpu_sc as plsc

B = 4096
CON = 13
CAT = 26
V = 100000
D = 32
ROWS = 1 + CON + CAT

_INFO = plsc.get_sparse_core_info()
NC, NS, L = _INFO.num_cores, _INFO.num_subcores, _INFO.num_lanes
NW = NC * NS
BPW = B // NW
NG = BPW // L


def _sc_kernel(tab_hbm, xcat_hbm, xcon_hbm, wcon_hbm, bcon_hbm, cls_hbm,
               out_hbm,
               idx_v, gbuf_v, dbuf_v, xc_v, w_v, bb_v, cls_v, gsem):
    wid = lax.axis_index("s") * NC + lax.axis_index("c")
    b0 = pl.multiple_of(wid * BPW, BPW)

    def fire_body(f, carry):
        pltpu.sync_copy(xcat_hbm.at[f, pl.ds(b0, BPW)], idx_v.at[f])
        for d in range(D):
            pltpu.async_copy(tab_hbm.at[f, d].at[idx_v.at[f]],
                             gbuf_v.at[f * D + d], gsem)
        return carry

    lax.fori_loop(0, CAT, fire_body, 0)

    for j in range(CON):
        pltpu.sync_copy(xcon_hbm.at[j, pl.ds(b0, BPW)], xc_v.at[j])
    pltpu.sync_copy(cls_hbm, cls_v)

    for d in range(D):
        sp = cls_v[pl.ds(d * L, L)]
        for g in range(NG):
            dbuf_v[d, pl.ds(g * L, L)] = sp
    pltpu.sync_copy(dbuf_v, out_hbm.at[pl.ds(0, D), pl.ds(b0, BPW)])

    def con_body(j, carry):
        xg = [xc_v[j, pl.ds(g * L, L)] for g in range(NG)]
        jo = pl.multiple_of(j * D * L, D * L)
        pltpu.sync_copy(wcon_hbm.at[pl.ds(jo, D * L)], w_v)
        pltpu.sync_copy(bcon_hbm.at[pl.ds(jo, D * L)], bb_v)
        for d in range(D):
            w_s = w_v[pl.ds(d * L, L)]
            b_s = bb_v[pl.ds(d * L, L)]
            for g in range(NG):
                dbuf_v[d, pl.ds(g * L, L)] = jnp.maximum(
                    xg[g] * w_s + b_s, 0.0)
        r0 = pl.multiple_of((1 + j) * D, D)
        pltpu.sync_copy(dbuf_v, out_hbm.at[pl.ds(r0, D), pl.ds(b0, BPW)])
        return carry

    lax.fori_loop(0, CON, con_body, 0)

    def drain_body(f, carry):
        pltpu.make_async_copy(tab_hbm.at[0, pl.ds(0, D), pl.ds(0, BPW)],
                              gbuf_v.at[pl.ds(0, D)], gsem).wait()
        return carry

    lax.fori_loop(0, CAT, drain_body, 0)

    def write_body(f, carry):
        fo = pl.multiple_of(f * D, D)
        r0 = pl.multiple_of((1 + CON + f) * D, D)
        pltpu.sync_copy(gbuf_v.at[pl.ds(fo, D)],
                        out_hbm.at[pl.ds(r0, D), pl.ds(b0, BPW)])
        return carry

    lax.fori_loop(0, CAT, write_body, 0)


@functools.partial(
    pl.kernel,
    out_type=jax.ShapeDtypeStruct((ROWS * D, B), jnp.float32),
    mesh=plsc.VectorSubcoreMesh(core_axis_name="c", subcore_axis_name="s"),
    compiler_params=pltpu.CompilerParams(use_tc_tiling_on_sc=False),
    scratch_types=[
        pltpu.VMEM((CAT, BPW), jnp.int32),
        pltpu.VMEM((CAT * D, BPW), jnp.float32),
        pltpu.VMEM((D, BPW), jnp.float32),
        pltpu.VMEM((CON, BPW), jnp.float32),
        pltpu.VMEM((D * L,), jnp.float32),
        pltpu.VMEM((D * L,), jnp.float32),
        pltpu.VMEM((D * L,), jnp.float32),
        pltpu.SemaphoreType.DMA,
    ],
)
def _graph_embed(*refs):
    _sc_kernel(*refs)


def kernel(x_con, x_cat, cls, W_con, b_con, tables):
    tabT = jnp.transpose(tables, (0, 2, 1))
    wsp = jnp.broadcast_to(W_con[:, :, None], (CON, D, L)).reshape(-1)
    bsp = jnp.broadcast_to(b_con[:, :, None], (CON, D, L)).reshape(-1)
    csp = jnp.broadcast_to(cls[0, :, None], (D, L)).reshape(-1)
    out = _graph_embed(tabT, x_cat.T, x_con.T, wsp, bsp, csp)
    return out.reshape(ROWS, D, B).transpose(2, 0, 1)
